# baseline (speedup 1.0000x reference)
import functools

import jax
import jax.numpy as jnp
import numpy as np
from jax import lax
from jax.experimental import pallas as pl
from jax.experimental.pallas import tpu as pltpu
from jax.sharding import Mesh, NamedSharding, PartitionSpec as P

_S = 4096
_C = _S // 128


def _make_tc_body(rows):
    def _tc_body(xc_ref, yr_ref, rowsum_ref, colmin_out_ref,
                 xb_ref, yb_ref, colmin_ref, rmin_ref):
        for c in range(3):
            yb_ref[16 * c:16 * c + 16, :] = jnp.broadcast_to(
                yr_ref[c:c + 1, :], (16, _S)).astype(jnp.bfloat16)
        colmin_ref[...] = jnp.full((16, _S), jnp.inf, dtype=jnp.bfloat16)

        for c in range(3):
            xb_ref[c * rows:(c + 1) * rows, :] = jnp.broadcast_to(
                xc_ref[:, c:c + 1], (rows, 128)).astype(jnp.bfloat16)

        def row_group(r, _):
            base = r * 16
            xb0 = xb_ref[pl.ds(base, 16), :]
            xb1 = xb_ref[pl.ds(rows + base, 16), :]
            xb2 = xb_ref[pl.ds(2 * rows + base, 16), :]
            rmin = jnp.full((16, 128), jnp.inf, dtype=jnp.bfloat16)
            for c in range(_C):
                sl = slice(c * 128, (c + 1) * 128)
                dx = xb0 - yb_ref[0:16, sl]
                dy = xb1 - yb_ref[16:32, sl]
                dz = xb2 - yb_ref[32:48, sl]
                d2 = dx * dx + dy * dy + dz * dz
                rmin = jnp.minimum(rmin, d2)
                colmin_ref[:, sl] = jnp.minimum(colmin_ref[:, sl], d2)
            rmin_ref[pl.ds(base, 16), :] = rmin
            return 0

        lax.fori_loop(0, rows // 16, row_group, 0)

        row_d2 = jnp.min(rmin_ref[...], axis=1).astype(jnp.float32)
        rowsum_ref[...] = jnp.full((1, 1), jnp.sum(jnp.sqrt(row_d2)),
                                   dtype=jnp.float32)
        colmin_out_ref[...] = jnp.min(
            colmin_ref[...], axis=0, keepdims=True).astype(jnp.float32)

    return _tc_body


def _tc_partials(Xc, Yr, rows):
    return pl.pallas_call(
        _make_tc_body(rows),
        out_shape=[
            jax.ShapeDtypeStruct((1, 1), jnp.float32),
            jax.ShapeDtypeStruct((1, _S), jnp.float32),
        ],
        scratch_shapes=[
            pltpu.VMEM((3 * rows, 128), jnp.bfloat16),
            pltpu.VMEM((48, _S), jnp.bfloat16),
            pltpu.VMEM((16, _S), jnp.bfloat16),
            pltpu.VMEM((rows, 128), jnp.bfloat16),
        ],
    )(Xc, Yr)


def _merge_body(rowsums_ref, colmins_ref, out_ref):
    rowsum = jnp.sum(rowsums_ref[...])
    col_d2 = jnp.min(colmins_ref[...], axis=0, keepdims=True)
    loss2 = jnp.sum(jnp.sqrt(col_d2)) / _S
    out_ref[...] = jnp.full((1, 1), rowsum / _S + loss2, dtype=jnp.float32)


def _merge(rowsums, colmins):
    return pl.pallas_call(
        _merge_body,
        out_shape=jax.ShapeDtypeStruct((1, 1), jnp.float32),
    )(rowsums, colmins)


_DEVS = jax.devices()


def kernel(X, Y):
    Xc = X[0]
    Yr = jnp.transpose(Y[0], (1, 0))
    if len(_DEVS) < 2:
        s, col = _tc_partials(Xc, Yr, _S)
        out = _merge(s, col)
        return out[0, 0]

    mesh = Mesh(np.array(_DEVS[:2]), ("d",))
    half = functools.partial(_tc_partials, rows=_S // 2)
    f = jax.shard_map(half, mesh=mesh,
                      in_specs=(P("d", None), P(None, None)),
                      out_specs=(P("d", None), P("d", None)),
                      check_vma=False)
    Xs = jax.device_put(Xc, NamedSharding(mesh, P("d", None)))
    Yrep = jax.device_put(Yr, NamedSharding(mesh, P(None, None)))
    rowsums, colmins = f(Xs, Yrep)
    rowsums = jax.device_put(rowsums, NamedSharding(mesh, P(None, None)))
    colmins = jax.device_put(colmins, NamedSharding(mesh, P(None, None)))
    fm = jax.shard_map(_merge, mesh=mesh,
                       in_specs=(P(None, None), P(None, None)),
                       out_specs=P(None, None), check_vma=False)
    out = fm(rowsums, colmins)
    return out[0, 0]


# speedup vs baseline: 12.7568x; 12.7568x over previous
import jax
import jax.numpy as jnp
from jax import lax
from jax.experimental import pallas as pl
from jax.experimental.pallas import tpu as pltpu

_S = 4096
_RG = _S // 16
_C = _S // 128


def _chamfer_body(xc_ref, yc_ref, out_ref, xb_ref, yb_ref, colmin_ref,
                  rmin_ref):
    yr = jnp.transpose(yc_ref[...], (1, 0))
    for c in range(3):
        yb_ref[16 * c:16 * c + 16, :] = jnp.broadcast_to(
            yr[c:c + 1, :], (16, _S)).astype(jnp.bfloat16)
    colmin_ref[...] = jnp.full((16, _S), jnp.inf, dtype=jnp.bfloat16)

    for c in range(3):
        xb_ref[c * _S:(c + 1) * _S, :] = jnp.broadcast_to(
            xc_ref[:, c:c + 1], (_S, 128)).astype(jnp.bfloat16)

    def row_group(r, _):
        base = r * 16
        xb0 = xb_ref[pl.ds(base, 16), :]
        xb1 = xb_ref[pl.ds(_S + base, 16), :]
        xb2 = xb_ref[pl.ds(2 * _S + base, 16), :]
        rmin = jnp.full((16, 128), jnp.inf, dtype=jnp.bfloat16)
        for c in range(_C):
            sl = slice(c * 128, (c + 1) * 128)
            dx = xb0 - yb_ref[0:16, sl]
            dy = xb1 - yb_ref[16:32, sl]
            dz = xb2 - yb_ref[32:48, sl]
            d2 = dx * dx + dy * dy + dz * dz
            rmin = jnp.minimum(rmin, d2)
            colmin_ref[:, sl] = jnp.minimum(colmin_ref[:, sl], d2)
        rmin_ref[pl.ds(base, 16), :] = rmin
        return 0

    lax.fori_loop(0, _RG, row_group, 0)

    row_d2 = jnp.min(rmin_ref[...], axis=1).astype(jnp.float32)
    loss1 = jnp.sum(jnp.sqrt(row_d2)) / _S
    col_d2 = jnp.min(colmin_ref[...], axis=0,
                     keepdims=True).astype(jnp.float32)
    loss2 = jnp.sum(jnp.sqrt(col_d2)) / _S
    out_ref[...] = jnp.full((1, 1), loss1 + loss2, dtype=jnp.float32)


def kernel(X, Y):
    out = pl.pallas_call(
        _chamfer_body,
        out_shape=jax.ShapeDtypeStruct((1, 1), jnp.float32),
        scratch_shapes=[
            pltpu.VMEM((3 * _S, 128), jnp.bfloat16),
            pltpu.VMEM((48, _S), jnp.bfloat16),
            pltpu.VMEM((16, _S), jnp.bfloat16),
            pltpu.VMEM((_S, 128), jnp.bfloat16),
        ],
    )(X[0], Y[0])
    return out[0, 0]
